# baseline (speedup 1.0000x reference)
import jax
import jax.numpy as jnp
from jax import lax
from jax.experimental import pallas as pl
from jax.experimental.pallas import tpu as pltpu

_B = 1024
_C = 64
_T = 512
_V = 1000
_BB = 16


def _film_params_kernel(idx_ref, emb_ref, wsc_ref, bsc_ref, wsh_ref, bsh_ref,
                        out_ref):
    idx = idx_ref[0, :]
    iota = lax.broadcasted_iota(jnp.int32, (_B, _V), 1)
    onehot = (idx[:, None] == iota).astype(jnp.float32)
    emb = jnp.dot(onehot, emb_ref[...], preferred_element_type=jnp.float32)
    scale = lax.dot_general(emb, wsc_ref[...], (((1,), (1,)), ((), ())),
                            preferred_element_type=jnp.float32)
    shift = lax.dot_general(emb, wsh_ref[...], (((1,), (1,)), ((), ())),
                            preferred_element_type=jnp.float32)
    out_ref[:, :_C] = scale + bsc_ref[...]
    out_ref[:, _C:] = shift + bsh_ref[...]


def _film_apply_kernel(sh_ref, eeg_ref, out_ref):
    sh = sh_ref[...]
    scale = sh[:, :_C]
    shift = sh[:, _C:]
    out_ref[...] = (eeg_ref[...] * (1.0 + scale[:, :, None])
                    + shift[:, :, None])


def kernel(eeg, subject_idx, emb_table, W_scale, b_scale, W_shift, b_shift):
    idx = subject_idx.astype(jnp.int32).reshape(1, _B)
    bsc = b_scale.reshape(1, _C)
    bsh = b_shift.reshape(1, _C)

    sh = pl.pallas_call(
        _film_params_kernel,
        out_shape=jax.ShapeDtypeStruct((_B, 2 * _C), jnp.float32),
    )(idx, emb_table, W_scale, bsc, W_shift, bsh)

    out = pl.pallas_call(
        _film_apply_kernel,
        grid=(_B // _BB,),
        in_specs=[
            pl.BlockSpec((_BB, 2 * _C), lambda i: (i, 0)),
            pl.BlockSpec((_BB, _C, _T), lambda i: (i, 0, 0)),
        ],
        out_specs=pl.BlockSpec((_BB, _C, _T), lambda i: (i, 0, 0)),
        out_shape=jax.ShapeDtypeStruct((_B, _C, _T), jnp.float32),
        compiler_params=pltpu.CompilerParams(
            dimension_semantics=("arbitrary",)),
    )(sh, eeg)
    return out


# speedup vs baseline: 1.1255x; 1.1255x over previous
import jax
import jax.numpy as jnp
from jax import lax
from jax.experimental import pallas as pl
from jax.experimental.pallas import tpu as pltpu

_B = 1024
_C = 64
_T = 512
_V = 1000
_BB = 32


def _film_params_kernel(idx_ref, emb_ref, wsc_ref, bsc_ref, wsh_ref, bsh_ref,
                        out_ref):
    idx = idx_ref[0, :]
    iota = lax.broadcasted_iota(jnp.int32, (_B, _V), 1)
    onehot = (idx[:, None] == iota).astype(jnp.float32)
    emb = jnp.dot(onehot, emb_ref[...], preferred_element_type=jnp.float32)
    scale = lax.dot_general(emb, wsc_ref[...], (((1,), (1,)), ((), ())),
                            preferred_element_type=jnp.float32)
    shift = lax.dot_general(emb, wsh_ref[...], (((1,), (1,)), ((), ())),
                            preferred_element_type=jnp.float32)
    out_ref[:, :_C] = scale + bsc_ref[...]
    out_ref[:, _C:] = shift + bsh_ref[...]


def _film_apply_kernel(sh_ref, eeg_ref, out_ref):
    sh = sh_ref[...]
    scale = sh[:, :_C]
    shift = sh[:, _C:]
    out_ref[...] = (eeg_ref[...] * (1.0 + scale[:, :, None])
                    + shift[:, :, None])


def kernel(eeg, subject_idx, emb_table, W_scale, b_scale, W_shift, b_shift):
    idx = subject_idx.astype(jnp.int32).reshape(1, _B)
    bsc = b_scale.reshape(1, _C)
    bsh = b_shift.reshape(1, _C)

    sh = pl.pallas_call(
        _film_params_kernel,
        out_shape=jax.ShapeDtypeStruct((_B, 2 * _C), jnp.float32),
    )(idx, emb_table, W_scale, bsc, W_shift, bsh)

    out = pl.pallas_call(
        _film_apply_kernel,
        grid=(_B // _BB,),
        in_specs=[
            pl.BlockSpec((_BB, 2 * _C), lambda i: (i, 0)),
            pl.BlockSpec((_BB, _C, _T), lambda i: (i, 0, 0)),
        ],
        out_specs=pl.BlockSpec((_BB, _C, _T), lambda i: (i, 0, 0)),
        out_shape=jax.ShapeDtypeStruct((_B, _C, _T), jnp.float32),
        compiler_params=pltpu.CompilerParams(
            dimension_semantics=("arbitrary",)),
    )(sh, eeg)
    return out


# speedup vs baseline: 1.1472x; 1.0192x over previous
import jax
import jax.numpy as jnp
from jax import lax
from jax.experimental import pallas as pl
from jax.experimental.pallas import tpu as pltpu

_B = 1024
_C = 64
_T = 512
_V = 1000
_BB = 64


def _film_params_kernel(idx_ref, emb_ref, wsc_ref, bsc_ref, wsh_ref, bsh_ref,
                        out_ref):
    idx = idx_ref[0, :]
    iota = lax.broadcasted_iota(jnp.int32, (_B, _V), 1)
    onehot = (idx[:, None] == iota).astype(jnp.float32)
    emb = jnp.dot(onehot, emb_ref[...], preferred_element_type=jnp.float32)
    scale = lax.dot_general(emb, wsc_ref[...], (((1,), (1,)), ((), ())),
                            preferred_element_type=jnp.float32)
    shift = lax.dot_general(emb, wsh_ref[...], (((1,), (1,)), ((), ())),
                            preferred_element_type=jnp.float32)
    out_ref[:, :_C] = scale + bsc_ref[...]
    out_ref[:, _C:] = shift + bsh_ref[...]


def _film_apply_kernel(sh_ref, eeg_ref, out_ref):
    sh = sh_ref[...]
    scale = sh[:, :_C]
    shift = sh[:, _C:]
    out_ref[...] = (eeg_ref[...] * (1.0 + scale[:, :, None])
                    + shift[:, :, None])


def kernel(eeg, subject_idx, emb_table, W_scale, b_scale, W_shift, b_shift):
    idx = subject_idx.astype(jnp.int32).reshape(1, _B)
    bsc = b_scale.reshape(1, _C)
    bsh = b_shift.reshape(1, _C)

    sh = pl.pallas_call(
        _film_params_kernel,
        out_shape=jax.ShapeDtypeStruct((_B, 2 * _C), jnp.float32),
    )(idx, emb_table, W_scale, bsc, W_shift, bsh)

    out = pl.pallas_call(
        _film_apply_kernel,
        grid=(_B // _BB,),
        in_specs=[
            pl.BlockSpec((_BB, 2 * _C), lambda i: (i, 0)),
            pl.BlockSpec((_BB, _C, _T), lambda i: (i, 0, 0)),
        ],
        out_specs=pl.BlockSpec((_BB, _C, _T), lambda i: (i, 0, 0)),
        out_shape=jax.ShapeDtypeStruct((_B, _C, _T), jnp.float32),
        compiler_params=pltpu.CompilerParams(
            dimension_semantics=("arbitrary",)),
    )(sh, eeg)
    return out
